# baseline (device time: 501091 ns/iter reference)
import jax
import jax.numpy as jnp
from jax import lax
from jax.experimental import pallas as pl
from jax.experimental.pallas import tpu as pltpu

N_DEV = 4
H = 8
DH = 128
SQ = 2048
SKV_SHARD = 2048
QT = 128
N_QT = SQ // QT
import os
ABLATE_AR = os.environ.get("ABLATE_AR") == "1"
ABLATE_COMPUTE = os.environ.get("ABLATE_COMPUTE") == "1"

SCALE = 0.08838834764831843
FIXED_MAX = 12.0
BF16 = jnp.bfloat16
MESH = pl.DeviceIdType.MESH


def _body(x_ref, wq_ref, wo_ref, kt_hbm, vt_hbm, out_ref,
          kfull, vfull, rsbuf,
          ksend, krecv, vsend, vrecv, rssend, rsrecv, agsend, agrecv):
    me = lax.axis_index("i")

    bar = pltpu.get_barrier_semaphore()
    peers = [lax.rem(me + d, N_DEV) for d in range(1, N_DEV)]
    for peer in peers:
        pl.semaphore_signal(bar, inc=1, device_id=(peer,),
                            device_id_type=MESH)
    pl.semaphore_wait(bar, N_DEV - 1)

    kv_rdmas = []
    for peer in peers:
        for src_hbm, full, ssem, rsem in (
            (kt_hbm, kfull, ksend, krecv),
            (vt_hbm, vfull, vsend, vrecv),
        ):
            r = pltpu.make_async_remote_copy(
                src_ref=src_hbm.at[pl.ds(H * peer, H)],
                dst_ref=full.at[me],
                send_sem=ssem.at[peer],
                recv_sem=rsem.at[me],
                device_id=(peer,),
                device_id_type=MESH,
            )
            r.start()
            kv_rdmas.append(r)

    kl = pltpu.make_async_copy(kt_hbm.at[pl.ds(H * me, H)],
                               kfull.at[me], krecv.at[me])
    kl.start()
    vl = pltpu.make_async_copy(vt_hbm.at[pl.ds(H * me, H)],
                               vfull.at[me], vrecv.at[me])
    vl.start()
    kl.wait()
    vl.wait()

    for peer in peers:
        for full, ssem, rsem in ((kfull, ksend, krecv), (vfull, vsend, vrecv)):
            pltpu.make_async_remote_copy(
                src_ref=kt_hbm.at[pl.ds(0, H)] if full is kfull
                else vt_hbm.at[pl.ds(0, H)],
                dst_ref=full.at[peer],
                send_sem=ssem.at[peer],
                recv_sem=rsem.at[peer],
                device_id=(me,),
                device_id_type=MESH,
            ).wait_recv()

    def _compute():
        for rho in range(3):
            r = (3 - rho) % 3
            kbs = [kb for kb in range(128) if kb % 3 == r]
            if r != 0:
                kbs = [0] + kbs
            n_qb = 11 if rho < 2 else 10
            for hb in (0, 4):
                ksels = [jnp.concatenate(
                    [kfull[kb // 32, hb + hh, pl.ds((kb % 32) * 64, 64)]
                     for kb in kbs], axis=0) for hh in range(4)]
                vsels = [jnp.concatenate(
                    [vfull[kb // 32, hb + hh, pl.ds((kb % 32) * 64, 64)]
                     for kb in kbs], axis=0) for hh in range(4)]

                def qb_body(j, c2, rho=rho, hb=hb, ksels=ksels, vsels=vsels):
                    qb = 3 * j + rho
                    q4 = jnp.dot(x_ref[qb], wq_ref[:, hb * DH:(hb + 4) * DH],
                                 preferred_element_type=jnp.float32)
                    q4 = (q4 * SCALE).astype(BF16)
                    ctxs = []
                    for hh in range(4):
                        qs = q4[:, hh * DH:(hh + 1) * DH]
                        s = lax.dot_general(
                            qs, ksels[hh], (((1,), (1,)), ((), ())),
                            preferred_element_type=jnp.float32)
                        p = jnp.exp(s - FIXED_MAX)
                        l = p.sum(axis=1, keepdims=True)
                        ctx = lax.dot_general(
                            p.astype(BF16), vsels[hh], (((1,), (0,)), ((), ())),
                            preferred_element_type=jnp.float32)
                        if rho != 0:
                            kd = kfull[0, hb + hh, pl.ds(qb * 64, 64)]
                            vd = vfull[0, hb + hh, pl.ds(qb * 64, 64)]
                            sd = lax.dot_general(
                                qs, kd, (((1,), (1,)), ((), ())),
                                preferred_element_type=jnp.float32)
                            pd = jnp.exp(sd - FIXED_MAX)
                            l = l + pd.sum(axis=1, keepdims=True)
                            ctx = ctx + lax.dot_general(
                                pd.astype(BF16), vd, (((1,), (0,)), ((), ())),
                                preferred_element_type=jnp.float32)
                        ctxs.append((ctx / l).astype(BF16))
                    ctx4 = jnp.concatenate(ctxs, axis=1)
                    contrib = jnp.dot(ctx4, wo_ref[hb * DH:(hb + 4) * DH, :],
                                      preferred_element_type=jnp.float32)
                    if hb == 0:
                        out_ref[qb] = contrib
                    else:
                        out_ref[qb] = out_ref[qb] + contrib
                    return c2

                lax.fori_loop(0, n_qb, qb_body, 0)

    if not ABLATE_COMPUTE:
        _compute()

    for r in kv_rdmas:
        r.wait_send()

    if ABLATE_AR:
        return

    myq = lax.rem(me + 1, N_DEV)
    rs_rdmas = []
    for d in range(1, N_DEV):
        peer = peers[d - 1]
        pq = lax.rem(peer + 1, N_DEV)
        r = pltpu.make_async_remote_copy(
            src_ref=out_ref.at[pl.ds(pq * 8, 8)],
            dst_ref=rsbuf.at[N_DEV - 1 - d],
            send_sem=rssend.at[peer],
            recv_sem=rsrecv.at[me],
            device_id=(peer,),
            device_id_type=MESH,
        )
        r.start()
        rs_rdmas.append(r)
    for d in range(1, N_DEV):
        peer = peers[d - 1]
        pltpu.make_async_remote_copy(
            src_ref=out_ref.at[pl.ds(0, 8)],
            dst_ref=rsbuf.at[d - 1],
            send_sem=rssend.at[peer],
            recv_sem=rsrecv.at[peer],
            device_id=(me,),
            device_id_type=MESH,
        ).wait_recv()

    acc_q = out_ref[pl.ds(myq * 8, 8)]
    for j in range(N_DEV - 1):
        acc_q = acc_q + rsbuf[j]
    out_ref[pl.ds(myq * 8, 8)] = acc_q

    ag_rdmas = []
    for peer in peers:
        r = pltpu.make_async_remote_copy(
            src_ref=out_ref.at[pl.ds(myq * 8, 8)],
            dst_ref=out_ref.at[pl.ds(myq * 8, 8)],
            send_sem=agsend.at[peer],
            recv_sem=agrecv.at[me],
            device_id=(peer,),
            device_id_type=MESH,
        )
        r.start()
        ag_rdmas.append(r)
    for peer in peers:
        pq = lax.rem(peer + 1, N_DEV)
        pltpu.make_async_remote_copy(
            src_ref=out_ref.at[pl.ds(0, 8)],
            dst_ref=out_ref.at[pl.ds(pq * 8, 8)],
            send_sem=agsend.at[peer],
            recv_sem=agrecv.at[peer],
            device_id=(me,),
            device_id_type=MESH,
        ).wait_recv()

    for r in rs_rdmas:
        r.wait_send()
    for r in ag_rdmas:
        r.wait_send()


def kernel(x, Wq, K_ext, V_ext, Wo):
    x2 = x[0].astype(BF16).reshape(2 * N_QT, 64, 1024)
    Wq2 = Wq.astype(BF16)
    Wo2 = Wo.astype(BF16)
    Kt = K_ext[0].transpose(1, 0, 2).astype(BF16)
    Vt = V_ext[0].transpose(1, 0, 2).astype(BF16)

    out = pl.pallas_call(
        _body,
        out_shape=jax.ShapeDtypeStruct((2 * N_QT, 64, 1024), jnp.float32),
        in_specs=[
            pl.BlockSpec(memory_space=pltpu.VMEM),
            pl.BlockSpec(memory_space=pltpu.VMEM),
            pl.BlockSpec(memory_space=pltpu.VMEM),
            pl.BlockSpec(memory_space=pl.ANY),
            pl.BlockSpec(memory_space=pl.ANY),
        ],
        out_specs=pl.BlockSpec(memory_space=pltpu.VMEM),
        scratch_shapes=[
            pltpu.VMEM((N_DEV, H, SKV_SHARD, DH), BF16),
            pltpu.VMEM((N_DEV, H, SKV_SHARD, DH), BF16),
            pltpu.VMEM((N_DEV - 1, 8, 64, 1024), jnp.float32),
            pltpu.SemaphoreType.DMA((N_DEV,)),
            pltpu.SemaphoreType.DMA((N_DEV,)),
            pltpu.SemaphoreType.DMA((N_DEV,)),
            pltpu.SemaphoreType.DMA((N_DEV,)),
            pltpu.SemaphoreType.DMA((N_DEV,)),
            pltpu.SemaphoreType.DMA((N_DEV,)),
            pltpu.SemaphoreType.DMA((N_DEV,)),
            pltpu.SemaphoreType.DMA((N_DEV,)),
        ],
        compiler_params=pltpu.CompilerParams(
            collective_id=0, vmem_limit_bytes=63 * 1024 * 1024),
    )(x2, Wq2, Wo2, Kt, Vt)
    return out.reshape(1, SQ, 1024)


# device time: 455915 ns/iter; 1.0991x vs baseline; 1.0991x over previous
import jax
import jax.numpy as jnp
from jax import lax
from jax.experimental import pallas as pl
from jax.experimental.pallas import tpu as pltpu

N_DEV = 4
H = 8
DH = 128
SQ = 2048
SKV_SHARD = 2048
QT = 128
N_QT = SQ // QT
import os
ABLATE_AR = os.environ.get("ABLATE_AR") == "1"
ABLATE_COMPUTE = os.environ.get("ABLATE_COMPUTE") == "1"

SCALE = 0.08838834764831843
FIXED_MAX = 12.0
BF16 = jnp.bfloat16
MESH = pl.DeviceIdType.MESH


def _body(x_ref, wq_ref, wo_ref, kt_hbm, vt_hbm, out_ref,
          kfull, vfull, rsbuf, outbf,
          ksend, krecv, vsend, vrecv, rssend, rsrecv, agsend, agrecv):
    me = lax.axis_index("i")

    bar = pltpu.get_barrier_semaphore()
    peers = [lax.rem(me + d, N_DEV) for d in range(1, N_DEV)]
    for peer in peers:
        pl.semaphore_signal(bar, inc=1, device_id=(peer,),
                            device_id_type=MESH)
    pl.semaphore_wait(bar, N_DEV - 1)

    kv_rdmas = []
    for peer in peers:
        for src_hbm, full, ssem, rsem in (
            (kt_hbm, kfull, ksend, krecv),
            (vt_hbm, vfull, vsend, vrecv),
        ):
            r = pltpu.make_async_remote_copy(
                src_ref=src_hbm.at[pl.ds(H * peer, H)],
                dst_ref=full.at[me],
                send_sem=ssem.at[peer],
                recv_sem=rsem.at[me],
                device_id=(peer,),
                device_id_type=MESH,
            )
            r.start()
            kv_rdmas.append(r)

    kl = pltpu.make_async_copy(kt_hbm.at[pl.ds(H * me, H)],
                               kfull.at[me], krecv.at[me])
    kl.start()
    vl = pltpu.make_async_copy(vt_hbm.at[pl.ds(H * me, H)],
                               vfull.at[me], vrecv.at[me])
    vl.start()
    kl.wait()
    vl.wait()

    for peer in peers:
        for full, ssem, rsem in ((kfull, ksend, krecv), (vfull, vsend, vrecv)):
            pltpu.make_async_remote_copy(
                src_ref=kt_hbm.at[pl.ds(0, H)] if full is kfull
                else vt_hbm.at[pl.ds(0, H)],
                dst_ref=full.at[peer],
                send_sem=ssem.at[peer],
                recv_sem=rsem.at[peer],
                device_id=(me,),
                device_id_type=MESH,
            ).wait_recv()

    def _compute():
        for rho in range(3):
            r = (3 - rho) % 3
            kbs = [kb for kb in range(128) if kb % 3 == r]
            if r != 0:
                kbs = [0] + kbs
            n_qb = 11 if rho < 2 else 10
            for hb in (0, 4):
                ksels = [jnp.concatenate(
                    [kfull[kb // 32, hb + hh, pl.ds((kb % 32) * 64, 64)]
                     for kb in kbs], axis=0) for hh in range(4)]
                vsels = [jnp.concatenate(
                    [vfull[kb // 32, hb + hh, pl.ds((kb % 32) * 64, 64)]
                     for kb in kbs], axis=0) for hh in range(4)]

                def qb_body(j, c2, rho=rho, hb=hb, ksels=ksels, vsels=vsels):
                    qb = 3 * j + rho
                    q4 = jnp.dot(x_ref[qb], wq_ref[:, hb * DH:(hb + 4) * DH],
                                 preferred_element_type=jnp.float32)
                    q4 = (q4 * SCALE).astype(BF16)
                    ctxs = []
                    for hh in range(4):
                        qs = q4[:, hh * DH:(hh + 1) * DH]
                        s = lax.dot_general(
                            qs, ksels[hh], (((1,), (1,)), ((), ())),
                            preferred_element_type=jnp.float32)
                        p = jnp.exp(s - FIXED_MAX)
                        l = p.sum(axis=1, keepdims=True)
                        ctx = lax.dot_general(
                            p.astype(BF16), vsels[hh], (((1,), (0,)), ((), ())),
                            preferred_element_type=jnp.float32)
                        if rho != 0:
                            kd = kfull[0, hb + hh, pl.ds(qb * 64, 64)]
                            vd = vfull[0, hb + hh, pl.ds(qb * 64, 64)]
                            sd = lax.dot_general(
                                qs, kd, (((1,), (1,)), ((), ())),
                                preferred_element_type=jnp.float32)
                            pd = jnp.exp(sd - FIXED_MAX)
                            l = l + pd.sum(axis=1, keepdims=True)
                            ctx = ctx + lax.dot_general(
                                pd.astype(BF16), vd, (((1,), (0,)), ((), ())),
                                preferred_element_type=jnp.float32)
                        ctxs.append((ctx / l).astype(BF16))
                    ctx4 = jnp.concatenate(ctxs, axis=1)
                    contrib = jnp.dot(ctx4, wo_ref[hb * DH:(hb + 4) * DH, :],
                                      preferred_element_type=jnp.float32)
                    if hb == 0:
                        out_ref[qb] = contrib
                    else:
                        out_ref[qb] = out_ref[qb] + contrib
                    return c2

                lax.fori_loop(0, n_qb, qb_body, 0)

    if not ABLATE_COMPUTE:
        _compute()

    for r in kv_rdmas:
        r.wait_send()

    if ABLATE_AR:
        return

    for i in range(2 * N_QT):
        outbf[i] = out_ref[i].astype(BF16)

    myq = lax.rem(me + 1, N_DEV)
    rs_rdmas = []
    for d in range(1, N_DEV):
        peer = peers[d - 1]
        pq = lax.rem(peer + 1, N_DEV)
        r = pltpu.make_async_remote_copy(
            src_ref=outbf.at[pl.ds(pq * 8, 8)],
            dst_ref=rsbuf.at[N_DEV - 1 - d],
            send_sem=rssend.at[peer],
            recv_sem=rsrecv.at[me],
            device_id=(peer,),
            device_id_type=MESH,
        )
        r.start()
        rs_rdmas.append(r)
    for d in range(1, N_DEV):
        peer = peers[d - 1]
        pltpu.make_async_remote_copy(
            src_ref=outbf.at[pl.ds(0, 8)],
            dst_ref=rsbuf.at[d - 1],
            send_sem=rssend.at[peer],
            recv_sem=rsrecv.at[peer],
            device_id=(me,),
            device_id_type=MESH,
        ).wait_recv()

    acc_q = out_ref[pl.ds(myq * 8, 8)]
    for j in range(N_DEV - 1):
        acc_q = acc_q + rsbuf[j].astype(jnp.float32)
    out_ref[pl.ds(myq * 8, 8)] = acc_q
    outbf[pl.ds(myq * 8, 8)] = acc_q.astype(BF16)

    ag_rdmas = []
    for peer in peers:
        r = pltpu.make_async_remote_copy(
            src_ref=outbf.at[pl.ds(myq * 8, 8)],
            dst_ref=outbf.at[pl.ds(myq * 8, 8)],
            send_sem=agsend.at[peer],
            recv_sem=agrecv.at[me],
            device_id=(peer,),
            device_id_type=MESH,
        )
        r.start()
        ag_rdmas.append(r)
    for peer in peers:
        pq = lax.rem(peer + 1, N_DEV)
        pltpu.make_async_remote_copy(
            src_ref=outbf.at[pl.ds(0, 8)],
            dst_ref=outbf.at[pl.ds(pq * 8, 8)],
            send_sem=agsend.at[peer],
            recv_sem=agrecv.at[peer],
            device_id=(me,),
            device_id_type=MESH,
        ).wait_recv()
        out_ref[pl.ds(pq * 8, 8)] = outbf[pl.ds(pq * 8, 8)].astype(jnp.float32)

    for r in rs_rdmas:
        r.wait_send()
    for r in ag_rdmas:
        r.wait_send()


def kernel(x, Wq, K_ext, V_ext, Wo):
    x2 = x[0].astype(BF16).reshape(2 * N_QT, 64, 1024)
    Wq2 = Wq.astype(BF16)
    Wo2 = Wo.astype(BF16)
    Kt = K_ext[0].transpose(1, 0, 2).astype(BF16)
    Vt = V_ext[0].transpose(1, 0, 2).astype(BF16)

    out = pl.pallas_call(
        _body,
        out_shape=jax.ShapeDtypeStruct((2 * N_QT, 64, 1024), jnp.float32),
        in_specs=[
            pl.BlockSpec(memory_space=pltpu.VMEM),
            pl.BlockSpec(memory_space=pltpu.VMEM),
            pl.BlockSpec(memory_space=pltpu.VMEM),
            pl.BlockSpec(memory_space=pl.ANY),
            pl.BlockSpec(memory_space=pl.ANY),
        ],
        out_specs=pl.BlockSpec(memory_space=pltpu.VMEM),
        scratch_shapes=[
            pltpu.VMEM((N_DEV, H, SKV_SHARD, DH), BF16),
            pltpu.VMEM((N_DEV, H, SKV_SHARD, DH), BF16),
            pltpu.VMEM((N_DEV - 1, 8, 64, 1024), BF16),
            pltpu.VMEM((2 * N_QT, 64, 1024), BF16),
            pltpu.SemaphoreType.DMA((N_DEV,)),
            pltpu.SemaphoreType.DMA((N_DEV,)),
            pltpu.SemaphoreType.DMA((N_DEV,)),
            pltpu.SemaphoreType.DMA((N_DEV,)),
            pltpu.SemaphoreType.DMA((N_DEV,)),
            pltpu.SemaphoreType.DMA((N_DEV,)),
            pltpu.SemaphoreType.DMA((N_DEV,)),
            pltpu.SemaphoreType.DMA((N_DEV,)),
        ],
        compiler_params=pltpu.CompilerParams(
            collective_id=0, vmem_limit_bytes=63 * 1024 * 1024),
    )(x2, Wq2, Wo2, Kt, Vt)
    return out.reshape(1, SQ, 1024)
